# baseline (device time: 25253 ns/iter reference)
import jax
import jax.numpy as jnp
from jax import lax
from jax.experimental import pallas as pl
from jax.experimental.pallas import tpu as pltpu

NCHUNK = 8


def kernel(x, assign, W1, W2):
    t, d = x.shape
    e_loc, _, f = W1.shape
    rows = t // NCHUNK

    assign2d = assign.reshape(t, 1)

    def body(x_ref, a_ref, w1_ref, w2_ref, out_ref,
             xsend, xrecv, arecv, prt, rbuf, send_sems, recv_sems):
        my_x = lax.axis_index("x")
        my_y = lax.axis_index("y")
        my_z = lax.axis_index("z")
        peer = (my_x, 1 - my_y, my_z)

        barrier_sem = pltpu.get_barrier_semaphore()
        pl.semaphore_signal(barrier_sem, inc=1, device_id=peer,
                            device_id_type=pl.DeviceIdType.MESH)
        pl.semaphore_wait(barrier_sem, 1)

        rdma_a = pltpu.make_async_remote_copy(
            src_ref=a_ref, dst_ref=arecv,
            send_sem=send_sems.at[0], recv_sem=recv_sems.at[0],
            device_id=peer, device_id_type=pl.DeviceIdType.MESH)
        rdma_a.start()
        xsends = []
        for c in range(NCHUNK):
            sl = pl.ds(c * rows, rows)
            xsend[sl, :] = x_ref[sl, :].astype(jnp.bfloat16)
            r = pltpu.make_async_remote_copy(
                src_ref=xsend.at[sl], dst_ref=xrecv.at[sl],
                send_sem=send_sems.at[1 + c], recv_sem=recv_sems.at[1 + c],
                device_id=peer, device_id_type=pl.DeviceIdType.MESH)
            r.start()
            xsends.append(r)

        def moe(Xb, A):
            n = Xb.shape[0]
            acc = jnp.zeros((n, d), jnp.float32)
            for le in range(e_loc):
                e_glob = e_loc * my_y + le
                h = jnp.maximum(
                    jnp.dot(Xb, w1_ref[le].astype(jnp.bfloat16),
                            preferred_element_type=jnp.float32),
                    0.0).astype(jnp.bfloat16)
                o = jnp.dot(h, w2_ref[le].astype(jnp.bfloat16),
                            preferred_element_type=jnp.float32)
                acc = acc + jnp.where(A == e_glob, o, 0.0)
            return acc

        acc_m = moe(xsend[...], a_ref[...])

        rdma_a.wait_recv()
        psends = []
        for c in range(NCHUNK):
            sl = pl.ds(c * rows, rows)
            xsends[c].wait_recv()
            acc_p = moe(xrecv[sl, :], arecv[sl, :])
            prt[sl, :] = acc_p.astype(jnp.bfloat16)
            r = pltpu.make_async_remote_copy(
                src_ref=prt.at[sl], dst_ref=rbuf.at[sl],
                send_sem=send_sems.at[1 + NCHUNK + c],
                recv_sem=recv_sems.at[1 + NCHUNK + c],
                device_id=peer, device_id_type=pl.DeviceIdType.MESH)
            r.start()
            psends.append(r)

        for c in range(NCHUNK):
            sl = pl.ds(c * rows, rows)
            psends[c].wait_recv()
            out_ref[sl, :] = (acc_m[c * rows:(c + 1) * rows, :]
                              + rbuf[sl, :].astype(jnp.float32))

        rdma_a.wait_send()
        for r in xsends:
            r.wait_send()
        for r in psends:
            r.wait_send()

    nsem = 1 + 2 * NCHUNK
    return pl.pallas_call(
        body,
        out_shape=jax.ShapeDtypeStruct((t, d), jnp.float32),
        in_specs=[
            pl.BlockSpec(memory_space=pltpu.VMEM),
            pl.BlockSpec(memory_space=pltpu.VMEM),
            pl.BlockSpec(memory_space=pltpu.VMEM),
            pl.BlockSpec(memory_space=pltpu.VMEM),
        ],
        out_specs=pl.BlockSpec(memory_space=pltpu.VMEM),
        scratch_shapes=[
            pltpu.VMEM((t, d), jnp.bfloat16),
            pltpu.VMEM((t, d), jnp.bfloat16),
            pltpu.VMEM((t, 1), jnp.int32),
            pltpu.VMEM((t, d), jnp.bfloat16),
            pltpu.VMEM((t, d), jnp.bfloat16),
            pltpu.SemaphoreType.DMA((nsem,)),
            pltpu.SemaphoreType.DMA((nsem,)),
        ],
        compiler_params=pltpu.CompilerParams(collective_id=0),
    )(x, assign2d, W1, W2)


# device time: 25162 ns/iter; 1.0036x vs baseline; 1.0036x over previous
import jax
import jax.numpy as jnp
from jax import lax
from jax.experimental import pallas as pl
from jax.experimental.pallas import tpu as pltpu

NCHUNK = 4


def kernel(x, assign, W1, W2):
    t, d = x.shape
    e_loc, _, f = W1.shape
    rows = t // NCHUNK

    assign2d = assign.reshape(t, 1)

    def body(x_ref, a_ref, w1_ref, w2_ref, out_ref,
             xsend, xrecv, arecv, prt, rbuf, send_sems, recv_sems):
        my_x = lax.axis_index("x")
        my_y = lax.axis_index("y")
        my_z = lax.axis_index("z")
        peer = (my_x, 1 - my_y, my_z)

        barrier_sem = pltpu.get_barrier_semaphore()
        pl.semaphore_signal(barrier_sem, inc=1, device_id=peer,
                            device_id_type=pl.DeviceIdType.MESH)
        pl.semaphore_wait(barrier_sem, 1)

        rdma_a = pltpu.make_async_remote_copy(
            src_ref=a_ref, dst_ref=arecv,
            send_sem=send_sems.at[0], recv_sem=recv_sems.at[0],
            device_id=peer, device_id_type=pl.DeviceIdType.MESH)
        rdma_a.start()
        xsends = []
        for c in range(NCHUNK):
            sl = pl.ds(c * rows, rows)
            xsend[sl, :] = x_ref[sl, :].astype(jnp.bfloat16)
            r = pltpu.make_async_remote_copy(
                src_ref=xsend.at[sl], dst_ref=xrecv.at[sl],
                send_sem=send_sems.at[1 + c], recv_sem=recv_sems.at[1 + c],
                device_id=peer, device_id_type=pl.DeviceIdType.MESH)
            r.start()
            xsends.append(r)

        def moe(Xb, A):
            n = Xb.shape[0]
            acc = jnp.zeros((n, d), jnp.float32)
            for le in range(e_loc):
                e_glob = e_loc * my_y + le
                h = jnp.maximum(
                    jnp.dot(Xb, w1_ref[le].astype(jnp.bfloat16),
                            preferred_element_type=jnp.float32),
                    0.0).astype(jnp.bfloat16)
                o = jnp.dot(h, w2_ref[le].astype(jnp.bfloat16),
                            preferred_element_type=jnp.float32)
                acc = acc + jnp.where(A == e_glob, o, 0.0)
            return acc

        acc_m = moe(xsend[...], a_ref[...])

        rdma_a.wait_recv()
        psends = []
        for c in range(NCHUNK):
            sl = pl.ds(c * rows, rows)
            xsends[c].wait_recv()
            acc_p = moe(xrecv[sl, :], arecv[sl, :])
            prt[sl, :] = acc_p.astype(jnp.bfloat16)
            r = pltpu.make_async_remote_copy(
                src_ref=prt.at[sl], dst_ref=rbuf.at[sl],
                send_sem=send_sems.at[1 + NCHUNK + c],
                recv_sem=recv_sems.at[1 + NCHUNK + c],
                device_id=peer, device_id_type=pl.DeviceIdType.MESH)
            r.start()
            psends.append(r)

        for c in range(NCHUNK):
            sl = pl.ds(c * rows, rows)
            psends[c].wait_recv()
            out_ref[sl, :] = (acc_m[c * rows:(c + 1) * rows, :]
                              + rbuf[sl, :].astype(jnp.float32))

        rdma_a.wait_send()
        for r in xsends:
            r.wait_send()
        for r in psends:
            r.wait_send()

    nsem = 1 + 2 * NCHUNK
    return pl.pallas_call(
        body,
        out_shape=jax.ShapeDtypeStruct((t, d), jnp.float32),
        in_specs=[
            pl.BlockSpec(memory_space=pltpu.VMEM),
            pl.BlockSpec(memory_space=pltpu.VMEM),
            pl.BlockSpec(memory_space=pltpu.VMEM),
            pl.BlockSpec(memory_space=pltpu.VMEM),
        ],
        out_specs=pl.BlockSpec(memory_space=pltpu.VMEM),
        scratch_shapes=[
            pltpu.VMEM((t, d), jnp.bfloat16),
            pltpu.VMEM((t, d), jnp.bfloat16),
            pltpu.VMEM((t, 1), jnp.int32),
            pltpu.VMEM((t, d), jnp.bfloat16),
            pltpu.VMEM((t, d), jnp.bfloat16),
            pltpu.SemaphoreType.DMA((nsem,)),
            pltpu.SemaphoreType.DMA((nsem,)),
        ],
        compiler_params=pltpu.CompilerParams(collective_id=0),
    )(x, assign2d, W1, W2)
